# Initial kernel scaffold; baseline (speedup 1.0000x reference)
import functools

import jax
import jax.numpy as jnp
from jax import lax
from jax.experimental import pallas as pl
from jax.experimental.pallas import tpu as pltpu
from jax.experimental.pallas import tpu_sc as plsc

N_NODES = 10000
N_EDGES = 320000
DIM = 128
HID = 64

BLK = 400
GRID = N_NODES // BLK

NC = 2
NS = 16
NW = NC * NS
CHUNK = 80
EPT = N_EDGES // NW
NCHUNK = EPT // CHUNK
NROWS = N_EDGES // CHUNK
RPT = N_NODES // NS


def _pre_body(x_ref, w_ref, a12_ref, wh_ref, s12_ref, m_ref, mscr):
    i = pl.program_id(0)
    wh = jnp.dot(x_ref[...], w_ref[...], preferred_element_type=jnp.float32)
    wh_ref[...] = wh
    s12 = jnp.dot(wh, a12_ref[...], preferred_element_type=jnp.float32)
    s12_ref[...] = s12
    bm1 = jnp.max(s12[:, 0])
    bm2 = jnp.max(s12[:, 1])

    @pl.when(i == 0)
    def _():
        mscr[0] = bm1
        mscr[1] = bm2

    @pl.when(i > 0)
    def _():
        mscr[0] = jnp.maximum(mscr[0], bm1)
        mscr[1] = jnp.maximum(mscr[1], bm2)

    mz = mscr[0] + mscr[1]
    m_ref[...] = jnp.full((1, 16), jnp.maximum(mz, 0.2 * mz), jnp.float32)


_pre_call = pl.pallas_call(
    _pre_body,
    grid=(GRID,),
    in_specs=[
        pl.BlockSpec((BLK, DIM), lambda i: (i, 0)),
        pl.BlockSpec((DIM, DIM), lambda i: (0, 0)),
        pl.BlockSpec((DIM, 2), lambda i: (0, 0)),
    ],
    out_specs=[
        pl.BlockSpec((BLK, DIM), lambda i: (i, 0)),
        pl.BlockSpec((BLK, 2), lambda i: (i, 0)),
        pl.BlockSpec((1, 16), lambda i: (0, 0)),
    ],
    out_shape=[
        jax.ShapeDtypeStruct((N_NODES, DIM), jnp.float32),
        jax.ShapeDtypeStruct((N_NODES, 2), jnp.float32),
        jax.ShapeDtypeStruct((1, 16), jnp.float32),
    ],
    scratch_shapes=[pltpu.SMEM((2,), jnp.float32)],
)


def _sc_body(wh_hbm, s1_hbm, s2_hbm, m_hbm, src_hbm, tgt_hbm, zeros_hbm,
             part_hbm, psum_hbm,
             s1_v, s2_v, m_v, src_v, tgt_v, rows_v, p_v, acc_v, shared, sem):
    cid = lax.axis_index("c")
    sid = lax.axis_index("s")
    wid = cid * NS + sid

    pltpu.sync_copy(s1_hbm, s1_v)
    pltpu.sync_copy(s2_hbm, s2_v)
    pltpu.sync_copy(m_hbm, m_v)
    pltpu.sync_copy(src_hbm.at[pl.ds(wid * NCHUNK, NCHUNK)], src_v)
    pltpu.sync_copy(tgt_hbm.at[pl.ds(wid * NCHUNK, NCHUNK)], tgt_v)
    pltpu.sync_copy(zeros_hbm.at[pl.ds(sid * RPT, RPT)],
                    shared.at[pl.ds(sid * RPT, RPT)])
    plsc.subcore_barrier()

    mvec = m_v[...]

    def chunk_body(j, acc):
        pltpu.async_copy(wh_hbm.at[src_v.at[j]], rows_v, sem).wait()

        def score_step(i, a_in):
            s_idx = src_v[j, pl.ds(i * 16, 16)]
            t_idx = tgt_v[j, pl.ds(i * 16, 16)]
            e = plsc.load_gather(s1_v, [s_idx]) + plsc.load_gather(s2_v, [t_idx])
            e = jnp.where(e > 0, e, 0.2 * e)
            p = jnp.exp(e - mvec)
            p_v[pl.ds(i * 16, 16)] = p
            return a_in + p

        acc = lax.fori_loop(0, CHUNK // 16, score_step, acc)

        def scale_step(r, carry):
            pr = p_v[r]
            for g in range(DIM // 16):
                rows_v[r, pl.ds(g * 16, 16)] = rows_v[r, pl.ds(g * 16, 16)] * pr
            return carry

        lax.fori_loop(0, CHUNK, scale_step, 0)

        pltpu.sync_copy(rows_v, shared.at[tgt_v.at[j]], add=True)
        return acc

    acc = lax.fori_loop(0, NCHUNK, chunk_body, jnp.zeros((16,), jnp.float32))
    acc_v[...] = acc
    pltpu.sync_copy(acc_v, psum_hbm.at[wid])
    plsc.subcore_barrier()
    pltpu.sync_copy(shared.at[pl.ds(sid * RPT, RPT)],
                    part_hbm.at[cid, pl.ds(sid * RPT, RPT)])


_sc_call = functools.partial(
    pl.kernel,
    mesh=plsc.VectorSubcoreMesh(core_axis_name="c", subcore_axis_name="s"),
    out_type=[
        jax.ShapeDtypeStruct((NC, N_NODES, DIM), jnp.float32),
        jax.ShapeDtypeStruct((NW, 16), jnp.float32),
    ],
    scratch_types=[
        pltpu.VMEM((N_NODES,), jnp.float32),
        pltpu.VMEM((N_NODES,), jnp.float32),
        pltpu.VMEM((16,), jnp.float32),
        pltpu.VMEM((NCHUNK, CHUNK), jnp.int32),
        pltpu.VMEM((NCHUNK, CHUNK), jnp.int32),
        pltpu.VMEM((CHUNK, DIM), jnp.float32),
        pltpu.VMEM((CHUNK,), jnp.float32),
        pltpu.VMEM((16,), jnp.float32),
        pltpu.VMEM_SHARED((N_NODES, DIM), jnp.float32),
        pltpu.SemaphoreType.DMA,
    ],
)(_sc_body)


def _post_body(p0_ref, p1_ref, psum_ref, w1_ref, b1_ref, w2_ref, b2_ref,
               xnew_ref, ev_ref):
    z = jnp.sum(psum_ref[...])
    xn = (p0_ref[...] + p1_ref[...]) * (1.0 / z)
    xnew_ref[...] = xn
    h = jnp.maximum(
        jnp.dot(xn, w1_ref[...], preferred_element_type=jnp.float32)
        + b1_ref[...], 0.0)
    t = (jnp.dot(h, w2_ref[...], preferred_element_type=jnp.float32)
         + b2_ref[...])
    sp = jnp.where(t > 30.0, t, jnp.log(1.0 + jnp.exp(jnp.minimum(t, 30.0))))
    ev_ref[...] = sp + 1.0


_post_call = pl.pallas_call(
    _post_body,
    grid=(GRID,),
    in_specs=[
        pl.BlockSpec((BLK, DIM), lambda i: (i, 0)),
        pl.BlockSpec((BLK, DIM), lambda i: (i, 0)),
        pl.BlockSpec((NW, 16), lambda i: (0, 0)),
        pl.BlockSpec((DIM, HID), lambda i: (0, 0)),
        pl.BlockSpec((1, HID), lambda i: (0, 0)),
        pl.BlockSpec((HID, DIM), lambda i: (0, 0)),
        pl.BlockSpec((1, DIM), lambda i: (0, 0)),
    ],
    out_specs=[
        pl.BlockSpec((BLK, DIM), lambda i: (i, 0)),
        pl.BlockSpec((BLK, DIM), lambda i: (i, 0)),
    ],
    out_shape=[
        jax.ShapeDtypeStruct((N_NODES, DIM), jnp.float32),
        jax.ShapeDtypeStruct((N_NODES, DIM), jnp.float32),
    ],
)


def kernel(x, edge_index, W, a, W1, b1, W2, b2):
    a12 = jnp.concatenate([a[:DIM], a[DIM:]], axis=1)
    wh, s12, m16 = _pre_call(x, W, a12)

    src = edge_index[0].reshape(NROWS, CHUNK)
    tgt = edge_index[1].reshape(NROWS, CHUNK)
    zeros = jnp.zeros((N_NODES, DIM), jnp.float32)
    part, psum = _sc_call(wh, s12[:, 0], s12[:, 1], m16.reshape(16),
                          src, tgt, zeros)

    w2p = jnp.zeros((HID, DIM), jnp.float32).at[:, :3].set(W2)
    b2p = jnp.zeros((1, DIM), jnp.float32).at[0, :3].set(b2)
    xnew, evfull = _post_call(part[0], part[1], psum, W1,
                              b1.reshape(1, HID), w2p, b2p)
    return xnew, evfull[:, :3]


# speedup vs baseline: 6.9582x; 6.9582x over previous
import functools

import jax
import jax.numpy as jnp
from jax import lax
from jax.experimental import pallas as pl
from jax.experimental.pallas import tpu as pltpu
from jax.experimental.pallas import tpu_sc as plsc

N_NODES = 10000
N_EDGES = 320000
DIM = 128
HID = 64

BLK = 400
GRID = N_NODES // BLK

NC = 2
NS = 16
NW = NC * NS
CHUNK = 80
EPT = N_EDGES // NW
NCHUNK = EPT // CHUNK
NGRP = 5
GCHUNK = NCHUNK // NGRP
N_PAD = 10112
RPT = N_PAD // NS
BLKC = 632
GRIDC = N_PAD // BLKC


def _pre_body(x_ref, w_ref, a12_ref, wh_ref, s12_ref, m_ref, mscr):
    i = pl.program_id(0)
    wh = jnp.dot(x_ref[...], w_ref[...], preferred_element_type=jnp.float32)
    wh_ref[...] = wh
    s12 = jnp.dot(wh, a12_ref[...], preferred_element_type=jnp.float32)
    s12_ref[...] = s12
    bm1 = jnp.max(s12[:, 0])
    bm2 = jnp.max(s12[:, 1])

    @pl.when(i == 0)
    def _():
        mscr[0] = bm1
        mscr[1] = bm2

    @pl.when(i > 0)
    def _():
        mscr[0] = jnp.maximum(mscr[0], bm1)
        mscr[1] = jnp.maximum(mscr[1], bm2)

    mz = mscr[0] + mscr[1]
    m_ref[...] = jnp.full((1, 16), jnp.maximum(mz, 0.2 * mz), jnp.float32)


_pre_call = pl.pallas_call(
    _pre_body,
    grid=(GRID,),
    in_specs=[
        pl.BlockSpec((BLK, DIM), lambda i: (i, 0)),
        pl.BlockSpec((DIM, DIM), lambda i: (0, 0)),
        pl.BlockSpec((DIM, 2), lambda i: (0, 0)),
    ],
    out_specs=[
        pl.BlockSpec((BLK, DIM), lambda i: (i, 0)),
        pl.BlockSpec((BLK, 2), lambda i: (i, 0)),
        pl.BlockSpec((1, 16), lambda i: (0, 0)),
    ],
    out_shape=[
        jax.ShapeDtypeStruct((N_NODES, DIM), jnp.float32),
        jax.ShapeDtypeStruct((N_NODES, 2), jnp.float32),
        jax.ShapeDtypeStruct((1, 16), jnp.float32),
    ],
    scratch_shapes=[pltpu.SMEM((2,), jnp.float32)],
)


def _sc_body(wh_hbm, s1_hbm, s2_hbm, m_hbm, src_hbm, tgt_hbm, zeros_hbm,
             part_hbm, psum_hbm,
             s1_v, s2_v, m_v, src_v, tgt_v, rows_v, p_v, acc_v, shared, sem):
    cid = lax.axis_index("c")
    sid = lax.axis_index("s")
    wid = cid * NS + sid

    pltpu.sync_copy(s1_hbm, s1_v)
    pltpu.sync_copy(s2_hbm, s2_v)
    pltpu.sync_copy(m_hbm, m_v)
    pltpu.sync_copy(zeros_hbm.at[pl.ds(sid * RPT, RPT)],
                    shared.at[pl.ds(sid * RPT, RPT)])
    plsc.subcore_barrier()

    mvec = m_v[...]

    def group_body(g, acc_g):
        pltpu.sync_copy(src_hbm.at[wid, g], src_v)
        pltpu.sync_copy(tgt_hbm.at[wid, g], tgt_v)

        def chunk_body(j, acc):
            pltpu.async_copy(wh_hbm.at[src_v.at[j]], rows_v, sem).wait()

            def score_step(i, a_in):
                s_idx = src_v[j, pl.ds(i * 16, 16)]
                t_idx = tgt_v[j, pl.ds(i * 16, 16)]
                e = (plsc.load_gather(s1_v, [s_idx])
                     + plsc.load_gather(s2_v, [t_idx]))
                e = jnp.where(e > 0, e, 0.2 * e)
                p = jnp.exp(e - mvec)
                p_v[pl.ds(i * 16, 16)] = p
                return a_in + p

            acc = lax.fori_loop(0, CHUNK // 16, score_step, acc)

            def scale_step(b, carry):
                p16 = p_v[pl.ds(b * 16, 16)]
                for k in range(16):
                    r = b * 16 + k
                    pr = p16[k]
                    for col in range(DIM // 16):
                        rows_v[r, pl.ds(col * 16, 16)] = (
                            rows_v[r, pl.ds(col * 16, 16)] * pr)
                return carry

            lax.fori_loop(0, CHUNK // 16, scale_step, 0)

            pltpu.sync_copy(rows_v, shared.at[tgt_v.at[j]], add=True)
            return acc

        return lax.fori_loop(0, GCHUNK, chunk_body, acc_g)

    acc = lax.fori_loop(0, NGRP, group_body, jnp.zeros((16,), jnp.float32))
    acc_v[...] = acc
    pltpu.sync_copy(acc_v, psum_hbm.at[pl.ds(wid * 16, 16)])
    plsc.subcore_barrier()
    pltpu.sync_copy(shared.at[pl.ds(sid * RPT, RPT)],
                    part_hbm.at[cid, pl.ds(sid * RPT, RPT)])


_sc_call = functools.partial(
    pl.kernel,
    mesh=plsc.VectorSubcoreMesh(core_axis_name="c", subcore_axis_name="s"),
    compiler_params=pltpu.CompilerParams(needs_layout_passes=False),
    out_type=[
        jax.ShapeDtypeStruct((NC, N_PAD, DIM), jnp.float32),
        jax.ShapeDtypeStruct((NW * 16,), jnp.float32),
    ],
    scratch_types=[
        pltpu.VMEM((N_NODES,), jnp.float32),
        pltpu.VMEM((N_NODES,), jnp.float32),
        pltpu.VMEM((16,), jnp.float32),
        pltpu.VMEM((GCHUNK, CHUNK), jnp.int32),
        pltpu.VMEM((GCHUNK, CHUNK), jnp.int32),
        pltpu.VMEM((CHUNK, DIM), jnp.float32),
        pltpu.VMEM((CHUNK,), jnp.float32),
        pltpu.VMEM((16,), jnp.float32),
        pltpu.VMEM_SHARED((N_PAD, DIM), jnp.float32),
        pltpu.SemaphoreType.DMA,
    ],
)(_sc_body)


def _post_body(p0_ref, p1_ref, psum_ref, w1_ref, b1_ref, w2_ref, b2_ref,
               xnew_ref, ev_ref):
    z = jnp.sum(psum_ref[...])
    xn = (p0_ref[...] + p1_ref[...]) * (1.0 / z)
    xnew_ref[...] = xn
    h = jnp.maximum(
        jnp.dot(xn, w1_ref[...], preferred_element_type=jnp.float32)
        + b1_ref[...], 0.0)
    t = (jnp.dot(h, w2_ref[...], preferred_element_type=jnp.float32)
         + b2_ref[...])
    sp = jnp.where(t > 30.0, t, jnp.log(1.0 + jnp.exp(jnp.minimum(t, 30.0))))
    ev_ref[...] = sp + 1.0


_post_call = pl.pallas_call(
    _post_body,
    grid=(GRIDC,),
    in_specs=[
        pl.BlockSpec((BLKC, DIM), lambda i: (i, 0)),
        pl.BlockSpec((BLKC, DIM), lambda i: (i, 0)),
        pl.BlockSpec((NW, 16), lambda i: (0, 0)),
        pl.BlockSpec((DIM, HID), lambda i: (0, 0)),
        pl.BlockSpec((1, HID), lambda i: (0, 0)),
        pl.BlockSpec((HID, DIM), lambda i: (0, 0)),
        pl.BlockSpec((1, DIM), lambda i: (0, 0)),
    ],
    out_specs=[
        pl.BlockSpec((BLKC, DIM), lambda i: (i, 0)),
        pl.BlockSpec((BLKC, DIM), lambda i: (i, 0)),
    ],
    out_shape=[
        jax.ShapeDtypeStruct((N_PAD, DIM), jnp.float32),
        jax.ShapeDtypeStruct((N_PAD, DIM), jnp.float32),
    ],
)


def kernel(x, edge_index, W, a, W1, b1, W2, b2):
    a12 = jnp.concatenate([a[:DIM], a[DIM:]], axis=1)
    wh, s12, m16 = _pre_call(x, W, a12)

    src = edge_index[0].reshape(NW, NGRP, GCHUNK, CHUNK)
    tgt = edge_index[1].reshape(NW, NGRP, GCHUNK, CHUNK)
    zeros = jnp.zeros((N_PAD, DIM), jnp.float32)
    part, psum = _sc_call(wh, s12[:, 0], s12[:, 1], m16.reshape(16),
                          src, tgt, zeros)

    w2p = jnp.zeros((HID, DIM), jnp.float32).at[:, :3].set(W2)
    b2p = jnp.zeros((1, DIM), jnp.float32).at[0, :3].set(b2)
    xnew, evfull = _post_call(part[0], part[1], psum.reshape(NW, 16), W1,
                              b1.reshape(1, HID), w2p, b2p)
    return xnew[:N_NODES], evfull[:N_NODES, :3]


# speedup vs baseline: 9.9299x; 1.4271x over previous
import functools

import jax
import jax.numpy as jnp
from jax import lax
from jax.experimental import pallas as pl
from jax.experimental.pallas import tpu as pltpu
from jax.experimental.pallas import tpu_sc as plsc

N_NODES = 10000
N_EDGES = 320000
DIM = 128
HID = 64

BLK = 400
GRID = N_NODES // BLK

NC = 2
NS = 16
NW = NC * NS
CHUNK = 80
EPT = N_EDGES // NW
NCHUNK = EPT // CHUNK
NGRP = 5
GCHUNK = NCHUNK // NGRP
N_PAD = 10112
RPT = N_PAD // NS
BLKC = 632
GRIDC = N_PAD // BLKC


def _pre_body(x_ref, w_ref, a12_ref, wh_ref, s12_ref, m_ref, mscr):
    i = pl.program_id(0)
    wh = jnp.dot(x_ref[...], w_ref[...], preferred_element_type=jnp.float32)
    wh_ref[...] = wh
    s12 = jnp.dot(wh, a12_ref[...], preferred_element_type=jnp.float32)
    s12_ref[...] = s12
    bm1 = jnp.max(s12[:, 0])
    bm2 = jnp.max(s12[:, 1])

    @pl.when(i == 0)
    def _():
        mscr[0] = bm1
        mscr[1] = bm2

    @pl.when(i > 0)
    def _():
        mscr[0] = jnp.maximum(mscr[0], bm1)
        mscr[1] = jnp.maximum(mscr[1], bm2)

    mz = mscr[0] + mscr[1]
    m_ref[...] = jnp.full((1, 16), jnp.maximum(mz, 0.2 * mz), jnp.float32)


_pre_call = pl.pallas_call(
    _pre_body,
    grid=(GRID,),
    in_specs=[
        pl.BlockSpec((BLK, DIM), lambda i: (i, 0)),
        pl.BlockSpec((DIM, DIM), lambda i: (0, 0)),
        pl.BlockSpec((DIM, 2), lambda i: (0, 0)),
    ],
    out_specs=[
        pl.BlockSpec((BLK, DIM), lambda i: (i, 0)),
        pl.BlockSpec((BLK, 2), lambda i: (i, 0)),
        pl.BlockSpec((1, 16), lambda i: (0, 0)),
    ],
    out_shape=[
        jax.ShapeDtypeStruct((N_NODES, DIM), jnp.float32),
        jax.ShapeDtypeStruct((N_NODES, 2), jnp.float32),
        jax.ShapeDtypeStruct((1, 16), jnp.float32),
    ],
    scratch_shapes=[pltpu.SMEM((2,), jnp.float32)],
)


def _sc_body(wh_hbm, s1_hbm, s2_hbm, m_hbm, src_hbm, tgt_hbm, zeros_hbm,
             part_hbm, psum_hbm,
             s1_v, s2_v, m_v, src_v, tgt_v, rows_a, rows_b, p_v, acc_v,
             shared, sem_a, sem_b):
    cid = lax.axis_index("c")
    sid = lax.axis_index("s")
    wid = cid * NS + sid

    pltpu.sync_copy(s1_hbm, s1_v)
    pltpu.sync_copy(s2_hbm, s2_v)
    pltpu.sync_copy(m_hbm, m_v)
    pltpu.sync_copy(zeros_hbm.at[pl.ds(sid * RPT, RPT)],
                    shared.at[pl.ds(sid * RPT, RPT)])
    plsc.subcore_barrier()

    mvec = m_v[...]

    def process(j, rows, acc):
        def score_step(i, a_in):
            s_idx = src_v[j, pl.ds(i * 16, 16)]
            t_idx = tgt_v[j, pl.ds(i * 16, 16)]
            e = (plsc.load_gather(s1_v, [s_idx])
                 + plsc.load_gather(s2_v, [t_idx]))
            e = jnp.where(e > 0, e, 0.2 * e)
            p = jnp.exp(e - mvec)
            p_v[pl.ds(i * 16, 16)] = p
            return a_in + p

        acc = lax.fori_loop(0, CHUNK // 16, score_step, acc)

        def scale_step(b, carry):
            p16 = p_v[pl.ds(b * 16, 16)]
            for k in range(16):
                r = b * 16 + k
                pr = p16[k]
                for col in range(DIM // 16):
                    rows[r, pl.ds(col * 16, 16)] = (
                        rows[r, pl.ds(col * 16, 16)] * pr)
            return carry

        lax.fori_loop(0, CHUNK // 16, scale_step, 0)

        pltpu.sync_copy(rows, shared.at[tgt_v.at[j]], add=True)
        return acc

    def fire(j, rows, s):
        pltpu.async_copy(wh_hbm.at[src_v.at[j]], rows, s)

    def drain(j, rows, s):
        pltpu.make_async_copy(wh_hbm.at[src_v.at[j]], rows, s).wait()

    def group_body(g, acc_g):
        pltpu.sync_copy(src_hbm.at[wid, g], src_v)
        pltpu.sync_copy(tgt_hbm.at[wid, g], tgt_v)

        fire(0, rows_a, sem_a)

        def pair_body(t, acc):
            ja = 2 * t
            fire(ja + 1, rows_b, sem_b)
            drain(ja, rows_a, sem_a)
            acc = process(ja, rows_a, acc)
            fire(ja + 2, rows_a, sem_a)
            drain(ja + 1, rows_b, sem_b)
            return process(ja + 1, rows_b, acc)

        acc_g = lax.fori_loop(0, (GCHUNK - 1) // 2, pair_body, acc_g)
        drain(GCHUNK - 1, rows_a, sem_a)
        return process(GCHUNK - 1, rows_a, acc_g)

    acc = lax.fori_loop(0, NGRP, group_body, jnp.zeros((16,), jnp.float32))
    acc_v[...] = acc
    pltpu.sync_copy(acc_v, psum_hbm.at[pl.ds(wid * 16, 16)])
    plsc.subcore_barrier()
    pltpu.sync_copy(shared.at[pl.ds(sid * RPT, RPT)],
                    part_hbm.at[cid, pl.ds(sid * RPT, RPT)])


_sc_call = functools.partial(
    pl.kernel,
    mesh=plsc.VectorSubcoreMesh(core_axis_name="c", subcore_axis_name="s"),
    compiler_params=pltpu.CompilerParams(needs_layout_passes=False),
    out_type=[
        jax.ShapeDtypeStruct((NC, N_PAD, DIM), jnp.float32),
        jax.ShapeDtypeStruct((NW * 16,), jnp.float32),
    ],
    scratch_types=[
        pltpu.VMEM((N_NODES,), jnp.float32),
        pltpu.VMEM((N_NODES,), jnp.float32),
        pltpu.VMEM((16,), jnp.float32),
        pltpu.VMEM((GCHUNK, CHUNK), jnp.int32),
        pltpu.VMEM((GCHUNK, CHUNK), jnp.int32),
        pltpu.VMEM((CHUNK, DIM), jnp.float32),
        pltpu.VMEM((CHUNK, DIM), jnp.float32),
        pltpu.VMEM((CHUNK,), jnp.float32),
        pltpu.VMEM((16,), jnp.float32),
        pltpu.VMEM_SHARED((N_PAD, DIM), jnp.float32),
        pltpu.SemaphoreType.DMA,
        pltpu.SemaphoreType.DMA,
    ],
)(_sc_body)


def _post_body(p0_ref, p1_ref, psum_ref, w1_ref, b1_ref, w2_ref, b2_ref,
               xnew_ref, ev_ref):
    z = jnp.sum(psum_ref[...])
    xn = (p0_ref[...] + p1_ref[...]) * (1.0 / z)
    xnew_ref[...] = xn
    h = jnp.maximum(
        jnp.dot(xn, w1_ref[...], preferred_element_type=jnp.float32)
        + b1_ref[...], 0.0)
    t = (jnp.dot(h, w2_ref[...], preferred_element_type=jnp.float32)
         + b2_ref[...])
    sp = jnp.where(t > 30.0, t, jnp.log(1.0 + jnp.exp(jnp.minimum(t, 30.0))))
    ev_ref[...] = sp + 1.0


_post_call = pl.pallas_call(
    _post_body,
    grid=(GRIDC,),
    in_specs=[
        pl.BlockSpec((BLKC, DIM), lambda i: (i, 0)),
        pl.BlockSpec((BLKC, DIM), lambda i: (i, 0)),
        pl.BlockSpec((NW, 16), lambda i: (0, 0)),
        pl.BlockSpec((DIM, HID), lambda i: (0, 0)),
        pl.BlockSpec((1, HID), lambda i: (0, 0)),
        pl.BlockSpec((HID, DIM), lambda i: (0, 0)),
        pl.BlockSpec((1, DIM), lambda i: (0, 0)),
    ],
    out_specs=[
        pl.BlockSpec((BLKC, DIM), lambda i: (i, 0)),
        pl.BlockSpec((BLKC, DIM), lambda i: (i, 0)),
    ],
    out_shape=[
        jax.ShapeDtypeStruct((N_PAD, DIM), jnp.float32),
        jax.ShapeDtypeStruct((N_PAD, DIM), jnp.float32),
    ],
)


def kernel(x, edge_index, W, a, W1, b1, W2, b2):
    a12 = jnp.concatenate([a[:DIM], a[DIM:]], axis=1)
    wh, s12, m16 = _pre_call(x, W, a12)

    src = edge_index[0].reshape(NW, NGRP, GCHUNK, CHUNK)
    tgt = edge_index[1].reshape(NW, NGRP, GCHUNK, CHUNK)
    zeros = jnp.zeros((N_PAD, DIM), jnp.float32)
    part, psum = _sc_call(wh, s12[:, 0], s12[:, 1], m16.reshape(16),
                          src, tgt, zeros)

    w2p = jnp.zeros((HID, DIM), jnp.float32).at[:, :3].set(W2)
    b2p = jnp.zeros((1, DIM), jnp.float32).at[0, :3].set(b2)
    xnew, evfull = _post_call(part[0], part[1], psum.reshape(NW, 16), W1,
                              b1.reshape(1, HID), w2p, b2p)
    return xnew[:N_NODES], evfull[:N_NODES, :3]
